# Initial kernel scaffold; baseline (speedup 1.0000x reference)
import jax
import jax.numpy as jnp
from jax.experimental import pallas as pl


def kernel(x, edge_index, Wl1, Wr1, att1, b1, Wl2, Wr2, att2, b2):
    raise NotImplementedError("write your pallas kernel here")


# speedup vs baseline: 1.6478x; 1.6478x over previous
import functools

import jax
import jax.numpy as jnp
from jax.experimental import pallas as pl
from jax.experimental.pallas import tpu as pltpu

N_NODES = 10000
D = 128


def _mm2_body(x_ref, wl_ref, wr_ref, xl_ref, xr_ref):
    x = x_ref[...]
    xl_ref[...] = jnp.dot(x, wl_ref[...], preferred_element_type=jnp.float32)
    xr_ref[...] = jnp.dot(x, wr_ref[...], preferred_element_type=jnp.float32)


@functools.partial(jax.jit, static_argnames=("block",))
def _mm2(x, wl, wr, block=2000):
    n = x.shape[0]
    grid = (n // block,)
    return pl.pallas_call(
        _mm2_body,
        grid=grid,
        in_specs=[
            pl.BlockSpec((block, D), lambda i: (i, 0)),
            pl.BlockSpec((D, D), lambda i: (0, 0)),
            pl.BlockSpec((D, D), lambda i: (0, 0)),
        ],
        out_specs=[
            pl.BlockSpec((block, D), lambda i: (i, 0)),
            pl.BlockSpec((block, D), lambda i: (i, 0)),
        ],
        out_shape=[
            jax.ShapeDtypeStruct((n, D), jnp.float32),
            jax.ShapeDtypeStruct((n, D), jnp.float32),
        ],
    )(x, wl, wr)


def _gatv2_layer(x, src, dst, Wl, Wr, att, bias):
    xl, xr = _mm2(x, Wl, Wr)
    e = xl[src] + xr[dst]
    e = jnp.where(e > 0, e, 0.2 * e)
    logit = e @ att
    a = jnp.exp(logit)
    denom = jax.ops.segment_sum(a, dst, num_segments=N_NODES)
    alpha = a / (denom[dst] + 1e-16)
    out = jax.ops.segment_sum(alpha[:, None] * xl[src], dst, num_segments=N_NODES)
    return out + bias


def kernel(x, edge_index, Wl1, Wr1, att1, b1, Wl2, Wr2, att2, b2):
    src = edge_index[0].astype(jnp.int32)
    dst = edge_index[1].astype(jnp.int32)
    h = _gatv2_layer(x, src, dst, Wl1, Wr1, att1, b1)
    h = jax.nn.relu(h)
    out = _gatv2_layer(h, src, dst, Wl2, Wr2, att2, b2)
    return out


# speedup vs baseline: 8.1173x; 4.9260x over previous
import functools
import math

import jax
import jax.numpy as jnp
from jax import lax
from jax.experimental import pallas as pl
from jax.experimental.pallas import tpu as pltpu
from jax.experimental.pallas import tpu_sc as plsc

N_NODES = 10000
NP = 10112
D = 128
E = 320000
CHUNK = 128
NC, NS = 2, 16
NW = NC * NS
CPT = math.ceil(E / (CHUNK * NW))
E_PAD = CPT * NW * CHUNK
ROWS_PT = NP // NS


def _mm2_body(x_ref, wl_ref, wr_ref, xl_ref, xr_ref):
    x = x_ref[...]
    xl_ref[...] = jnp.dot(x, wl_ref[...], preferred_element_type=jnp.float32)
    xr_ref[...] = jnp.dot(x, wr_ref[...], preferred_element_type=jnp.float32)


def _mm2(x, wl, wr, block=1264):
    n = x.shape[0]
    return pl.pallas_call(
        _mm2_body,
        grid=(n // block,),
        in_specs=[
            pl.BlockSpec((block, D), lambda i: (i, 0)),
            pl.BlockSpec((D, D), lambda i: (0, 0)),
            pl.BlockSpec((D, D), lambda i: (0, 0)),
        ],
        out_specs=[
            pl.BlockSpec((block, D), lambda i: (i, 0)),
            pl.BlockSpec((block, D), lambda i: (i, 0)),
        ],
        out_shape=[
            jax.ShapeDtypeStruct((n, D), jnp.float32),
            jax.ShapeDtypeStruct((n, D), jnp.float32),
        ],
    )(x, wl, wr)


def _norm_mm_body(acc_ref, den_ref, b_ref, wl_ref, wr_ref, xl_ref, xr_ref):
    a = acc_ref[0] + acc_ref[1]
    d = jnp.sum(den_ref[...], axis=1, keepdims=True)
    h = a / (d + 1e-16) + b_ref[...]
    h = jnp.maximum(h, 0.0)
    xl_ref[...] = jnp.dot(h, wl_ref[...], preferred_element_type=jnp.float32)
    xr_ref[...] = jnp.dot(h, wr_ref[...], preferred_element_type=jnp.float32)


def _norm_mm(acc, den2, b, wl, wr, block=1264):
    return pl.pallas_call(
        _norm_mm_body,
        grid=(NP // block,),
        in_specs=[
            pl.BlockSpec((2, block, D), lambda i: (0, i, 0)),
            pl.BlockSpec((block, NW), lambda i: (i, 0)),
            pl.BlockSpec((1, D), lambda i: (0, 0)),
            pl.BlockSpec((D, D), lambda i: (0, 0)),
            pl.BlockSpec((D, D), lambda i: (0, 0)),
        ],
        out_specs=[
            pl.BlockSpec((block, D), lambda i: (i, 0)),
            pl.BlockSpec((block, D), lambda i: (i, 0)),
        ],
        out_shape=[
            jax.ShapeDtypeStruct((NP, D), jnp.float32),
            jax.ShapeDtypeStruct((NP, D), jnp.float32),
        ],
    )(acc, den2, b, wl, wr)


def _norm_out_body(acc_ref, den_ref, b_ref, out_ref):
    a = acc_ref[0] + acc_ref[1]
    d = jnp.sum(den_ref[...], axis=1, keepdims=True)
    out_ref[...] = a / (d + 1e-16) + b_ref[...]


def _norm_out(acc, den2, b, block=1264):
    return pl.pallas_call(
        _norm_out_body,
        grid=(NP // block,),
        in_specs=[
            pl.BlockSpec((2, block, D), lambda i: (0, i, 0)),
            pl.BlockSpec((block, NW), lambda i: (i, 0)),
            pl.BlockSpec((1, D), lambda i: (0, 0)),
        ],
        out_specs=pl.BlockSpec((block, D), lambda i: (i, 0)),
        out_shape=jax.ShapeDtypeStruct((NP, D), jnp.float32),
    )(acc, den2, b)


_SC_MESH = plsc.VectorSubcoreMesh(core_axis_name="c", subcore_axis_name="s")


@functools.partial(
    pl.kernel,
    out_type=[
        jax.ShapeDtypeStruct((NC, NP, D), jnp.float32),
        jax.ShapeDtypeStruct((NC, NS, NP), jnp.float32),
    ],
    mesh=_SC_MESH,
    compiler_params=pltpu.CompilerParams(needs_layout_passes=False),
    scratch_types=[
        pltpu.VMEM((CHUNK,), jnp.int32),
        pltpu.VMEM((CHUNK,), jnp.int32),
        pltpu.VMEM((CHUNK + 16,), jnp.int32),
        pltpu.VMEM((CHUNK, D), jnp.float32),
        pltpu.VMEM((CHUNK, D), jnp.float32),
        pltpu.VMEM((D,), jnp.float32),
        pltpu.VMEM((NP,), jnp.float32),
        pltpu.VMEM_SHARED((NP, D), jnp.float32),
        pltpu.SemaphoreType.DMA,
        pltpu.SemaphoreType.DMA,
    ],
)
def _edge_pass(xl_hbm, xr_hbm, src_hbm, dst_hbm, att_hbm, z128_hbm,
               acc_out, den_out,
               src_v, dst_v, dstp_v, xl_v, xr_v, att_v, den_v,
               acc_s, sem1, sem2):
    c = lax.axis_index("c")
    s = lax.axis_index("s")
    wid = s * NC + c

    r0 = s * ROWS_PT
    pltpu.sync_copy(z128_hbm.at[pl.ds(r0, ROWS_PT)], acc_s.at[pl.ds(r0, ROWS_PT)])
    pltpu.sync_copy(att_hbm, att_v)
    zero16 = jnp.zeros((16,), jnp.float32)

    def zero_body(i, carry):
        den_v[pl.ds(i * 16, 16)] = zero16
        return carry

    lax.fori_loop(0, NP // 16, zero_body, 0)
    plsc.subcore_barrier()

    att_regs = [att_v[pl.ds(k * 16, 16)] for k in range(D // 16)]
    lane0 = lax.iota(jnp.int32, 16) == 0

    def chunk_body(t, carry):
        base = (wid * CPT + t) * CHUNK
        pltpu.sync_copy(src_hbm.at[pl.ds(base, CHUNK)], src_v)
        pltpu.sync_copy(dst_hbm.at[pl.ds(base, CHUNK)], dst_v)
        pltpu.sync_copy(dst_hbm.at[pl.ds(base, CHUNK)], dstp_v.at[pl.ds(0, CHUNK)])
        cp1 = pltpu.async_copy(xl_hbm.at[src_v], xl_v, sem1)
        cp2 = pltpu.async_copy(xr_hbm.at[dst_v], xr_v, sem2)
        cp1.wait()
        cp2.wait()

        def edge_body(e, carry2):
            accv = jnp.zeros((16,), jnp.float32)
            xls = []
            for k in range(D // 16):
                xlk = xl_v[e, pl.ds(k * 16, 16)]
                xrk = xr_v[e, pl.ds(k * 16, 16)]
                sk = xlk + xrk
                lk = jnp.where(sk > 0, sk, 0.2 * sk)
                accv = accv + lk * att_regs[k]
                xls.append(xlk)
            av = jnp.exp(jnp.full((16,), jnp.sum(accv), jnp.float32))
            for k in range(D // 16):
                xl_v[e, pl.ds(k * 16, 16)] = xls[k] * av
            d_e = jnp.full((16,), dstp_v[pl.ds(e, 16)][0], jnp.int32)
            plsc.addupdate_scatter(den_v, [d_e], av, mask=lane0)
            return carry2

        lax.fori_loop(0, CHUNK, edge_body, 0)
        pltpu.sync_copy(xl_v, acc_s.at[dst_v], add=True)
        return carry

    lax.fori_loop(0, CPT, chunk_body, 0)

    plsc.subcore_barrier()
    pltpu.sync_copy(acc_s.at[pl.ds(r0, ROWS_PT)], acc_out.at[c, pl.ds(r0, ROWS_PT)])
    pltpu.sync_copy(den_v, den_out.at[c, s])


def kernel(x, edge_index, Wl1, Wr1, att1, b1, Wl2, Wr2, att2, b2):
    src = edge_index[0].astype(jnp.int32)
    dst = edge_index[1].astype(jnp.int32)
    npad = E_PAD - E
    pad_idx = N_NODES + jnp.arange(npad, dtype=jnp.int32) % (NP - N_NODES)
    srcp = jnp.concatenate([src, pad_idx])
    dstp = jnp.concatenate([dst, pad_idx])
    x_pad = jnp.pad(x, ((0, NP - N_NODES), (0, 0)))
    z128 = jnp.zeros((NP, D), jnp.float32)

    xl1, xr1 = _mm2(x_pad, Wl1, Wr1)
    acc1, den1 = _edge_pass(xl1, xr1, srcp, dstp, att1, z128)
    den1t = den1.reshape(NW, NP).T
    xl2, xr2 = _norm_mm(acc1, den1t, b1.reshape(1, D), Wl2, Wr2)
    acc2, den2 = _edge_pass(xl2, xr2, srcp, dstp, att2, z128)
    den2t = den2.reshape(NW, NP).T
    out = _norm_out(acc2, den2t, b2.reshape(1, D))
    return out[:N_NODES]


# speedup vs baseline: 13.2100x; 1.6274x over previous
import functools
import math

import jax
import jax.numpy as jnp
from jax import lax
from jax.experimental import pallas as pl
from jax.experimental.pallas import tpu as pltpu
from jax.experimental.pallas import tpu_sc as plsc

N_NODES = 10000
NP = 10112
D = 128
E = 320000
CHUNK = 64
NC, NS = 2, 16
NW = NC * NS
WIN = 16
CPT = WIN * math.ceil(E / (WIN * CHUNK * NW))
E_PAD = CPT * NW * CHUNK
ROWS_PT = NP // NS


def _mm2_body(x_ref, wl_ref, wr_ref, xl_ref, xr_ref):
    x = x_ref[...]
    xl_ref[...] = jnp.dot(x, wl_ref[...], preferred_element_type=jnp.float32)
    xr_ref[...] = jnp.dot(x, wr_ref[...], preferred_element_type=jnp.float32)


def _mm2(x, wl, wr, block=1264):
    n = x.shape[0]
    return pl.pallas_call(
        _mm2_body,
        grid=(n // block,),
        in_specs=[
            pl.BlockSpec((block, D), lambda i: (i, 0)),
            pl.BlockSpec((D, D), lambda i: (0, 0)),
            pl.BlockSpec((D, D), lambda i: (0, 0)),
        ],
        out_specs=[
            pl.BlockSpec((block, D), lambda i: (i, 0)),
            pl.BlockSpec((block, D), lambda i: (i, 0)),
        ],
        out_shape=[
            jax.ShapeDtypeStruct((n, D), jnp.float32),
            jax.ShapeDtypeStruct((n, D), jnp.float32),
        ],
    )(x, wl, wr)


def _norm_mm_body(acc_ref, den_ref, b_ref, wl_ref, wr_ref, xl_ref, xr_ref):
    a = acc_ref[0] + acc_ref[1]
    d = jnp.sum(den_ref[...], axis=1, keepdims=True)
    h = a / (d + 1e-16) + b_ref[...]
    h = jnp.maximum(h, 0.0)
    xl_ref[...] = jnp.dot(h, wl_ref[...], preferred_element_type=jnp.float32)
    xr_ref[...] = jnp.dot(h, wr_ref[...], preferred_element_type=jnp.float32)


def _norm_mm(acc, den2, b, wl, wr, block=1264):
    return pl.pallas_call(
        _norm_mm_body,
        grid=(NP // block,),
        in_specs=[
            pl.BlockSpec((2, block, D), lambda i: (0, i, 0)),
            pl.BlockSpec((block, 8), lambda i: (i, 0)),
            pl.BlockSpec((1, D), lambda i: (0, 0)),
            pl.BlockSpec((D, D), lambda i: (0, 0)),
            pl.BlockSpec((D, D), lambda i: (0, 0)),
        ],
        out_specs=[
            pl.BlockSpec((block, D), lambda i: (i, 0)),
            pl.BlockSpec((block, D), lambda i: (i, 0)),
        ],
        out_shape=[
            jax.ShapeDtypeStruct((NP, D), jnp.float32),
            jax.ShapeDtypeStruct((NP, D), jnp.float32),
        ],
    )(acc, den2, b, wl, wr)


def _norm_out_body(acc_ref, den_ref, b_ref, out_ref):
    a = acc_ref[0] + acc_ref[1]
    d = jnp.sum(den_ref[...], axis=1, keepdims=True)
    out_ref[...] = a / (d + 1e-16) + b_ref[...]


def _norm_out(acc, den2, b, block=1264):
    return pl.pallas_call(
        _norm_out_body,
        grid=(NP // block,),
        in_specs=[
            pl.BlockSpec((2, block, D), lambda i: (0, i, 0)),
            pl.BlockSpec((block, 8), lambda i: (i, 0)),
            pl.BlockSpec((1, D), lambda i: (0, 0)),
        ],
        out_specs=pl.BlockSpec((block, D), lambda i: (i, 0)),
        out_shape=jax.ShapeDtypeStruct((NP, D), jnp.float32),
    )(acc, den2, b)


_SC_MESH = plsc.VectorSubcoreMesh(core_axis_name="c", subcore_axis_name="s")


@functools.partial(
    pl.kernel,
    out_type=[
        jax.ShapeDtypeStruct((NC, NP, D), jnp.float32),
        jax.ShapeDtypeStruct((NC, 1, NP), jnp.float32),
    ],
    mesh=_SC_MESH,
    compiler_params=pltpu.CompilerParams(needs_layout_passes=False),
    scratch_types=[
        pltpu.VMEM((2, WIN, CHUNK), jnp.int32),
        pltpu.VMEM((2, WIN, CHUNK), jnp.int32),
        pltpu.VMEM((2, CHUNK, D), jnp.float32),
        pltpu.VMEM((2, CHUNK, D), jnp.float32),
        pltpu.VMEM((2, CHUNK), jnp.float32),
        pltpu.VMEM((D,), jnp.float32),
        pltpu.VMEM_SHARED((NP, D), jnp.float32),
        pltpu.VMEM_SHARED((NP,), jnp.float32),
        pltpu.SemaphoreType.DMA,
        pltpu.SemaphoreType.DMA,
        pltpu.SemaphoreType.DMA,
        pltpu.SemaphoreType.DMA,
        pltpu.SemaphoreType.DMA,
    ],
)
def _edge_pass(xl_hbm, xr_hbm, src_hbm, dst_hbm, att_hbm, z128_hbm, z1_hbm,
               acc_out, den_out,
               src_v, dst_v, xl_b, xr_b, a_v, att_v,
               acc_s, den_s, sem_xl, sem_xr, sem_sc, sem_a, sem_ix):
    c = lax.axis_index("c")
    s = lax.axis_index("s")
    wid = s * NC + c

    r0 = s * ROWS_PT
    pltpu.sync_copy(z128_hbm.at[pl.ds(r0, ROWS_PT)], acc_s.at[pl.ds(r0, ROWS_PT)])

    @pl.when(s == 0)
    def _():
        pltpu.sync_copy(z1_hbm, den_s)
    pltpu.sync_copy(att_hbm, att_v)
    pltpu.sync_copy(src_hbm.at[wid, pl.ds(0, WIN)], src_v.at[0])
    pltpu.sync_copy(dst_hbm.at[wid, pl.ds(0, WIN)], dst_v.at[0])
    plsc.subcore_barrier()

    att_regs = [att_v[pl.ds(k * 16, 16)] for k in range(D // 16)]
    lanes = lax.iota(jnp.int32, 16)

    def gather_descs(tt):
        wb, row, b = (tt // WIN) % 2, tt % WIN, tt % 2
        return (
            pltpu.make_async_copy(xl_hbm.at[src_v.at[wb, row]], xl_b.at[b], sem_xl),
            pltpu.make_async_copy(xr_hbm.at[dst_v.at[wb, row]], xr_b.at[b], sem_xr),
        )

    def scatter_descs(tt):
        wb, row, b = (tt // WIN) % 2, tt % WIN, tt % 2
        return (
            pltpu.make_async_copy(xl_b.at[b], acc_s.at[dst_v.at[wb, row]], sem_sc),
            pltpu.make_async_copy(a_v.at[b], den_s.at[dst_v.at[wb, row]], sem_a),
        )

    def refill_descs(k):
        return (
            pltpu.make_async_copy(src_hbm.at[wid, pl.ds(k * WIN, WIN)],
                                  src_v.at[k % 2], sem_ix),
            pltpu.make_async_copy(dst_hbm.at[wid, pl.ds(k * WIN, WIN)],
                                  dst_v.at[k % 2], sem_ix),
        )

    def compute_chunk(tt):
        b = tt % 2

        def group_body(g, carry):
            pa = jnp.zeros((16,), jnp.float32)
            for j in range(16):
                e = g * 16 + j
                accv = jnp.zeros((16,), jnp.float32)
                xls = []
                for k in range(D // 16):
                    xlk = xl_b[b, e, pl.ds(k * 16, 16)]
                    xrk = xr_b[b, e, pl.ds(k * 16, 16)]
                    sk = xlk + xrk
                    lk = jnp.where(sk > 0, sk, 0.2 * sk)
                    accv = accv + lk * att_regs[k]
                    xls.append(xlk)
                av = jnp.exp(jnp.full((16,), jnp.sum(accv), jnp.float32))
                for k in range(D // 16):
                    xl_b[b, e, pl.ds(k * 16, 16)] = xls[k] * av
                pa = jnp.where(lanes == j, av, pa)
            a_v[b, pl.ds(g * 16, 16)] = pa
            return carry

        lax.fori_loop(0, CHUNK // 16, group_body, 0)

    for gd in gather_descs(0):
        gd.start()

    def body(t, carry):
        for gd in gather_descs(t):
            gd.wait()

        @pl.when(t >= 1)
        def _():
            for sd in scatter_descs(t - 1):
                sd.wait()

        @pl.when((t % WIN == WIN - 1) & (t <= CPT - 2))
        def _():
            for rd in refill_descs(t // WIN + 1):
                rd.wait()

        @pl.when(t <= CPT - 2)
        def _():
            for gd in gather_descs(t + 1):
                gd.start()

        @pl.when((t % WIN == 1) & (t <= CPT - WIN))
        def _():
            for rd in refill_descs(t // WIN + 1):
                rd.start()

        compute_chunk(t)
        rsd, asd = scatter_descs(t)
        rsd.start(add=True)
        asd.start(add=True)
        return carry

    lax.fori_loop(0, CPT, body, 0)

    for sd in scatter_descs(CPT - 1):
        sd.wait()
    plsc.subcore_barrier()
    pltpu.sync_copy(acc_s.at[pl.ds(r0, ROWS_PT)], acc_out.at[c, pl.ds(r0, ROWS_PT)])

    @pl.when(s == 0)
    def _():
        pltpu.sync_copy(den_s, den_out.at[c, 0])


def kernel(x, edge_index, Wl1, Wr1, att1, b1, Wl2, Wr2, att2, b2):
    src = edge_index[0].astype(jnp.int32)
    dst = edge_index[1].astype(jnp.int32)
    npad = E_PAD - E
    pad_idx = N_NODES + jnp.arange(npad, dtype=jnp.int32) % (NP - N_NODES)
    srcp = jnp.concatenate([src, pad_idx]).reshape(NW, CPT, CHUNK)
    dstp = jnp.concatenate([dst, pad_idx]).reshape(NW, CPT, CHUNK)
    x_pad = jnp.pad(x, ((0, NP - N_NODES), (0, 0)))
    z128 = jnp.zeros((NP, D), jnp.float32)
    z1 = jnp.zeros((NP,), jnp.float32)

    xl1, xr1 = _mm2(x_pad, Wl1, Wr1)
    acc1, den1 = _edge_pass(xl1, xr1, srcp, dstp, att1, z128, z1)
    den1t = jnp.pad(den1.reshape(NC, NP).T, ((0, 0), (0, 8 - NC)))
    xl2, xr2 = _norm_mm(acc1, den1t, b1.reshape(1, D), Wl2, Wr2)
    acc2, den2 = _edge_pass(xl2, xr2, srcp, dstp, att2, z128, z1)
    den2t = jnp.pad(den2.reshape(NC, NP).T, ((0, 0), (0, 8 - NC)))
    out = _norm_out(acc2, den2t, b2.reshape(1, D))
    return out[:N_NODES]
